# Initial kernel scaffold; baseline (speedup 1.0000x reference)
import jax
import jax.numpy as jnp
from jax.experimental import pallas as pl


def kernel(features, coords, w1, w2, w3, w4, w5, g1, g2, g3, g4, g5, bt1, bt2, bt3, bt4, bt5):
    raise NotImplementedError("write your pallas kernel here")


# speedup vs baseline: 9.7065x; 9.7065x over previous
import functools

import jax
import jax.numpy as jnp
from jax import lax
from jax.experimental import pallas as pl
from jax.experimental.pallas import tpu as pltpu
from jax.experimental.pallas import tpu_sc as plsc

B = 2
D = 16
H = 96
W = 96
C = 64
C2 = 2 * C
N = 80000
EPS = 1e-3

WP = 104
HP = 98
PR = HP * WP
P1 = D + 2
NP1 = B * P1
R1 = NP1 * PR
D2 = 7
P2 = D2 + 2
NP2 = B * P2
D3 = 3

NSC = 2
NTILE = 16
NW = NSC * NTILE
SCHUNK = 256
NCH = 10
NPW = NCH * SCHUNK
NPAD = NW * NPW


def _sc_scatter_body(feats, cb, cz, cy, cx, dense,
                     fbuf, bb, zb, yb, xb, di0, di1, sem):
    c = lax.axis_index("c")
    s = lax.axis_index("s")
    w = s * NSC + c
    dibs = [di0, di1]
    trash = R1 - 1
    lanes = lax.iota(jnp.int32, 16)
    for j in range(NCH):
        off = pl.multiple_of(w * NPW + j * SCHUNK, 8)
        pltpu.sync_copy(feats.at[pl.ds(off, SCHUNK)], fbuf)
        pltpu.sync_copy(cb.at[pl.ds(off, SCHUNK)], bb)
        pltpu.sync_copy(cz.at[pl.ds(off, SCHUNK)], zb)
        pltpu.sync_copy(cy.at[pl.ds(off, SCHUNK)], yb)
        pltpu.sync_copy(cx.at[pl.ds(off, SCHUNK)], xb)
        for i in range(SCHUNK // 16):
            bv = bb[pl.ds(i * 16, 16)]
            zv = zb[pl.ds(i * 16, 16)]
            yv = yb[pl.ds(i * 16, 16)]
            xv = xb[pl.ds(i * 16, 16)]
            dst = (bv * P1 + zv + 1) * PR + (yv + 1) * WP + (xv + 1)
            ok = (lanes + (off + i * 16)) < N
            dst = jnp.where(ok, dst, trash)
            dibs[i // 8][pl.ds((i % 8) * 16, 16)] = dst
        copies = [
            pltpu.async_copy(fbuf.at[pl.ds(q * 128, 128)],
                             dense.at[dibs[q]], sem)
            for q in range(SCHUNK // 128)
        ]
        for cp in copies:
            cp.wait()


def _sc_scatter(feats128, cb, cz, cy, cx):
    mesh = plsc.VectorSubcoreMesh(core_axis_name="c", subcore_axis_name="s",
                                  num_cores=NSC, num_subcores=NTILE)
    f = pl.kernel(
        _sc_scatter_body,
        out_type=(),
        mesh=mesh,
        scratch_types=[
            pltpu.VMEM((SCHUNK, C2), jnp.float32),
            pltpu.VMEM((SCHUNK,), jnp.int32),
            pltpu.VMEM((SCHUNK,), jnp.int32),
            pltpu.VMEM((SCHUNK,), jnp.int32),
            pltpu.VMEM((SCHUNK,), jnp.int32),
            pltpu.VMEM((128,), jnp.int32),
            pltpu.VMEM((128,), jnp.int32),
            pltpu.SemaphoreType.DMA,
        ],
    )
    dense_ref = jax.new_ref(jnp.zeros((R1, C2), jnp.float32))
    f(feats128, cb, cz, cy, cx, dense_ref)
    return dense_ref[...].reshape(NP1, PR, C2)


SLABN = 3 * PR
SPAD = 112
SLAB = SLABN + 2 * SPAD
IX3N = SLABN + 2 * WP
MEXT = PR + 2 * WP


def _bn_scale(g):
    return g[...] * (1.0 / jnp.sqrt(1.0 + EPS))


def _sub3_body(in0, in1, in2, wbig, g, bt, out, ix3, acc3, *, pads):
    ins = (in0, in1, in2)

    def compute():
        for kd in range(3):
            for kw in range(3):
                s = kd * PR + kw + SPAD - WP - 1
                col = pl.ds(kw * C, C)
                lo = min(max(0, SPAD - s), MEXT)
                if lo > 0:
                    ix3[pl.ds(0, lo), col] = jnp.zeros((lo, C), jnp.bfloat16)
                for p in range(3):
                    a = min(max(0, SPAD + p * PR - s), MEXT)
                    b = min(max(0, SPAD + (p + 1) * PR - s), MEXT)
                    if b > a:
                        ix3[pl.ds(a, b - a), col] = ins[p][
                            0, pl.ds(s + a - SPAD - p * PR, b - a),
                            pl.ds(0, C)].astype(jnp.bfloat16)
                hi = min(max(0, SPAD + 3 * PR - s), MEXT)
                if hi < MEXT:
                    ix3[pl.ds(hi, MEXT - hi), col] = jnp.zeros(
                        (MEXT - hi, C), jnp.bfloat16)
            if kd == 0:
                acc3[...] = jnp.dot(ix3[...], wbig[kd],
                                    preferred_element_type=jnp.float32)
            else:
                acc3[...] += jnp.dot(ix3[...], wbig[kd],
                                     preferred_element_type=jnp.float32)
        res = (acc3[pl.ds(0, PR), pl.ds(0, C)]
               + acc3[pl.ds(WP, PR), pl.ds(C, C)]
               + acc3[pl.ds(2 * WP, PR), pl.ds(2 * C, C)])
        msk = in1[0, :, C:].astype(jnp.float32)
        res = res * _bn_scale(g) + bt[...]
        res = jnp.maximum(res, 0.0) * msk
        out[0, :, :C] = res.astype(jnp.bfloat16)
        out[0, :, C:] = msk.astype(jnp.bfloat16)

    if pads is None:
        compute()
    else:
        per_b, plo, phi = pads
        i = pl.program_id(0)
        is_pad = (i % per_b == plo) | (i % per_b == phi)

        @pl.when(jnp.logical_not(is_pad))
        def _():
            compute()

        @pl.when(is_pad)
        def _():
            out[0] = jnp.zeros((PR, C2), jnp.bfloat16)


def _make_sub3(ppb, lo, hi, pads):
    if pads is None:
        per_b = hi - lo + 1

        def pmap(i, dk):
            return (i // per_b) * ppb + (i % per_b) + lo + dk - 1

        def omap(i):
            return ((i // per_b) * ppb + (i % per_b) + lo, 0, 0)

        body_pads = None
    else:
        per_b = ppb

        def pmap(i, dk):
            return ((i // per_b) * ppb
                    + jnp.clip((i % per_b) + dk - 1, 0, ppb - 1))

        def omap(i):
            return ((i // per_b) * ppb + (i % per_b), 0, 0)

        body_pads = (per_b, pads[0], pads[1])

    body = functools.partial(_sub3_body, pads=body_pads)

    def run(x, wbig, g, bt):
        return pl.pallas_call(
            body,
            grid=(B * per_b,),
            in_specs=[
                pl.BlockSpec((1, PR, C2), lambda i: (pmap(i, 0), 0, 0)),
                pl.BlockSpec((1, PR, C2), lambda i: (pmap(i, 1), 0, 0)),
                pl.BlockSpec((1, PR, C2), lambda i: (pmap(i, 2), 0, 0)),
                pl.BlockSpec((3, 3 * C, 3 * C), lambda i: (0, 0, 0)),
                pl.BlockSpec((1, C), lambda i: (0, 0)),
                pl.BlockSpec((1, C), lambda i: (0, 0)),
            ],
            out_specs=pl.BlockSpec((1, PR, C2), omap),
            out_shape=jax.ShapeDtypeStruct((x.shape[0], PR, C2), jnp.bfloat16),
            scratch_shapes=[
                pltpu.VMEM((MEXT, 3 * C), jnp.bfloat16),
                pltpu.VMEM((MEXT, 3 * C), jnp.float32),
            ],
        )(x, x, x, wbig, g, bt)

    return run


def _down2_body(in0, in1, in2, w3k, g, bt, out):
    i = pl.program_id(0)
    is_pad = (i % P2 == 0) | (i % P2 == P2 - 1)

    @pl.when(jnp.logical_not(is_pad))
    def _():
        acc = jnp.dot(in0[0, :, :C], w3k[0], preferred_element_type=jnp.float32)
        acc += jnp.dot(in1[0, :, :C], w3k[1], preferred_element_type=jnp.float32)
        acc += jnp.dot(in2[0, :, :C], w3k[2], preferred_element_type=jnp.float32)
        msum = (in0[0, :, C:].astype(jnp.float32)
                + in1[0, :, C:].astype(jnp.float32)
                + in2[0, :, C:].astype(jnp.float32))
        m = (msum > 0.0).astype(jnp.float32)
        res = acc * _bn_scale(g) + bt[...]
        res = jnp.maximum(res, 0.0) * m
        out[0, :, :C] = res.astype(jnp.bfloat16)
        out[0, :, C:] = m.astype(jnp.bfloat16)

    @pl.when(is_pad)
    def _():
        out[0] = jnp.zeros((PR, C2), jnp.bfloat16)


def _down_conv2(dense, w3k, g, bt):
    def in_map(dk):
        def f(i):
            b = i // P2
            p = jnp.clip(2 * (i % P2) - 1 + dk, 0, P1 - 1)
            return (b * P1 + p, 0, 0)
        return f

    return pl.pallas_call(
        _down2_body,
        grid=(B * P2,),
        in_specs=[
            pl.BlockSpec((1, PR, C2), in_map(0)),
            pl.BlockSpec((1, PR, C2), in_map(1)),
            pl.BlockSpec((1, PR, C2), in_map(2)),
            pl.BlockSpec((3, C, C), lambda i: (0, 0, 0)),
            pl.BlockSpec((1, C), lambda i: (0, 0)),
            pl.BlockSpec((1, C), lambda i: (0, 0)),
        ],
        out_specs=pl.BlockSpec((1, PR, C2), lambda i: (i, 0, 0)),
        out_shape=jax.ShapeDtypeStruct((NP2, PR, C2), jnp.bfloat16),
    )(dense, dense, dense, w3k, g, bt)


def _down5_body(in0, in1, in2, w3k, g, bt, out):
    acc = jnp.dot(in0[0, :, :C], w3k[0], preferred_element_type=jnp.float32)
    acc += jnp.dot(in1[0, :, :C], w3k[1], preferred_element_type=jnp.float32)
    acc += jnp.dot(in2[0, :, :C], w3k[2], preferred_element_type=jnp.float32)
    msum = (in0[0, :, C:].astype(jnp.float32)
            + in1[0, :, C:].astype(jnp.float32)
            + in2[0, :, C:].astype(jnp.float32))
    m = (msum > 0.0).astype(jnp.float32)
    res = acc * _bn_scale(g) + bt[...]
    out[0] = jnp.maximum(res, 0.0) * m


def _down_conv5(h4, w3k, g, bt):
    def in_map(dk):
        def f(i):
            b = i // D3
            return (b * P2 + 2 * (i % D3) + 1 + dk, 0, 0)
        return f

    return pl.pallas_call(
        _down5_body,
        grid=(B * D3,),
        in_specs=[
            pl.BlockSpec((1, PR, C2), in_map(0)),
            pl.BlockSpec((1, PR, C2), in_map(1)),
            pl.BlockSpec((1, PR, C2), in_map(2)),
            pl.BlockSpec((3, C, C), lambda i: (0, 0, 0)),
            pl.BlockSpec((1, C), lambda i: (0, 0)),
            pl.BlockSpec((1, C), lambda i: (0, 0)),
        ],
        out_specs=pl.BlockSpec((1, PR, C), lambda i: (i, 0, 0)),
        out_shape=jax.ShapeDtypeStruct((B * D3, PR, C), jnp.float32),
    )(h4, h4, h4, w3k, g, bt)


def _wbig(w):
    return jnp.transpose(w, (0, 2, 3, 1, 4)).reshape(3, 3 * C, 3 * C).astype(
        jnp.bfloat16)


def _conv_stack(dense, w1, w2, w3, w4, w5,
                g1, g2, g3, g4, g5, bt1, bt2, bt3, bt4, bt5):
    gs = [x.reshape(1, C) for x in (g1, g2, g3, g4, g5)]
    bts = [x.reshape(1, C) for x in (bt1, bt2, bt3, bt4, bt5)]

    sub1 = _make_sub3(P1, 1, D, None)
    h1 = sub1(dense, _wbig(w1), gs[0], bts[0])
    h2 = _down_conv2(h1, w2.reshape(3, C, C).astype(jnp.bfloat16),
                     gs[1], bts[1])
    sub3 = _make_sub3(P2, 0, 0, (0, P2 - 1))
    h3 = sub3(h2, _wbig(w3), gs[2], bts[2])
    sub4 = _make_sub3(P2, 1, D2, None)
    h4 = sub4(h3, _wbig(w4), gs[3], bts[3])
    h5 = _down_conv5(h4, w5.reshape(3, C, C).astype(jnp.bfloat16),
                     gs[4], bts[4])
    out = h5.reshape(B, D3, HP, WP, C)[:, :, 1:97, 1:97, :]
    return jnp.transpose(out, (0, 4, 1, 2, 3))


def kernel(features, coords, w1, w2, w3, w4, w5,
           g1, g2, g3, g4, g5, bt1, bt2, bt3, bt4, bt5):
    feats128 = jnp.pad(jnp.concatenate(
        [features, jnp.ones((N, C), jnp.float32)], axis=1),
        ((0, NPAD - N), (0, 0)))
    cpad = jnp.pad(coords, ((0, NPAD - N), (0, 0)))
    dense = _sc_scatter(feats128, cpad[:, 0], cpad[:, 1], cpad[:, 2],
                        cpad[:, 3])
    return _conv_stack(dense, w1, w2, w3, w4, w5,
                       g1, g2, g3, g4, g5, bt1, bt2, bt3, bt4, bt5)


# speedup vs baseline: 13.2073x; 1.3607x over previous
import functools

import jax
import jax.numpy as jnp
from jax import lax
from jax.experimental import pallas as pl
from jax.experimental.pallas import tpu as pltpu
from jax.experimental.pallas import tpu_sc as plsc

B = 2
D = 16
H = 96
W = 96
C = 64
C2 = 2 * C
N = 80000
EPS = 1e-3

WP = 104
HP = 98
PR = HP * WP
P1 = D + 2
NP1 = B * P1
R1 = NP1 * PR
D2 = 7
P2 = D2 + 2
NP2 = B * P2
D3 = 3

NSC = 2
NTILE = 16
NW = NSC * NTILE
SCHUNK = 256
NCH = 10
NPW = NCH * SCHUNK
NPAD = NW * NPW


def _sc_scatter_body(feats, cb, cz, cy, cx, dense,
                     fbuf, bb, zb, yb, xb, di0, di1, sem):
    c = lax.axis_index("c")
    s = lax.axis_index("s")
    w = s * NSC + c
    dibs = [di0, di1]
    trash = R1 - 1
    lanes = lax.iota(jnp.int32, 16)
    for j in range(NCH):
        off = pl.multiple_of(w * NPW + j * SCHUNK, 8)
        pltpu.sync_copy(feats.at[pl.ds(off, SCHUNK)], fbuf)
        pltpu.sync_copy(cb.at[pl.ds(off, SCHUNK)], bb)
        pltpu.sync_copy(cz.at[pl.ds(off, SCHUNK)], zb)
        pltpu.sync_copy(cy.at[pl.ds(off, SCHUNK)], yb)
        pltpu.sync_copy(cx.at[pl.ds(off, SCHUNK)], xb)
        for i in range(SCHUNK // 16):
            bv = bb[pl.ds(i * 16, 16)]
            zv = zb[pl.ds(i * 16, 16)]
            yv = yb[pl.ds(i * 16, 16)]
            xv = xb[pl.ds(i * 16, 16)]
            dst = (bv * P1 + zv + 1) * PR + (yv + 1) * WP + (xv + 1)
            ok = (lanes + (off + i * 16)) < N
            dst = jnp.where(ok, dst, trash)
            dibs[i // 8][pl.ds((i % 8) * 16, 16)] = dst
        copies = [
            pltpu.async_copy(fbuf.at[pl.ds(q * 128, 128)],
                             dense.at[dibs[q]], sem)
            for q in range(SCHUNK // 128)
        ]
        for cp in copies:
            cp.wait()


def _sc_scatter(feats128, cb, cz, cy, cx):
    mesh = plsc.VectorSubcoreMesh(core_axis_name="c", subcore_axis_name="s",
                                  num_cores=NSC, num_subcores=NTILE)
    f = pl.kernel(
        _sc_scatter_body,
        out_type=(),
        mesh=mesh,
        scratch_types=[
            pltpu.VMEM((SCHUNK, C2), jnp.float32),
            pltpu.VMEM((SCHUNK,), jnp.int32),
            pltpu.VMEM((SCHUNK,), jnp.int32),
            pltpu.VMEM((SCHUNK,), jnp.int32),
            pltpu.VMEM((SCHUNK,), jnp.int32),
            pltpu.VMEM((128,), jnp.int32),
            pltpu.VMEM((128,), jnp.int32),
            pltpu.SemaphoreType.DMA,
        ],
    )
    dense_ref = jax.new_ref(jnp.zeros((R1, C2), jnp.float32))
    f(feats128, cb, cz, cy, cx, dense_ref)
    return dense_ref[...].reshape(NP1, PR, C2)


SLABN = 3 * PR
SPAD = 112
SLAB = SLABN + 2 * SPAD
IX3N = SLABN + 2 * WP
MEXT = PR + 2 * WP
MHALF = 5200


def _bn_scale(g):
    return g[...] * (1.0 / jnp.sqrt(1.0 + EPS))


def _sub3_body(in1, in2, wbig, g, bt, out, ring, acc3, *, per_b, first, pads):
    i = pl.program_id(0)
    t = i % per_b
    s0 = (t + first + 2) % 3
    s1 = (t + first) % 3
    s2 = (t + first + 1) % 3

    def build_main(slot, src_prev, src_new):
        for kw in range(3):
            col = pl.ds(kw * C, C)
            h = WP + 1 - kw
            ring[slot, pl.ds(0, h), col] = src_prev[
                0, pl.ds(PR - h, h), pl.ds(0, C)].astype(jnp.bfloat16)
            ring[slot, pl.ds(h, PR), col] = src_new[
                0, pl.ds(0, PR), pl.ds(0, C)].astype(jnp.bfloat16)
            tl = MEXT - PR - h
            ring[slot, pl.ds(h + PR, tl), col] = jnp.zeros(
                (tl, C), jnp.bfloat16)

    def patch_tail(slot, src_next):
        for kw in range(3):
            col = pl.ds(kw * C, C)
            h = WP + 1 - kw
            tl = MEXT - PR - h
            ring[slot, pl.ds(h + PR, tl), col] = src_next[
                0, pl.ds(0, tl), pl.ds(0, C)].astype(jnp.bfloat16)

    def build_zero_plane(slot, src_next):
        for kw in range(3):
            col = pl.ds(kw * C, C)
            h = WP + 1 - kw
            ring[slot, pl.ds(0, h + PR), col] = jnp.zeros(
                (h + PR, C), jnp.bfloat16)
            tl = MEXT - PR - h
            ring[slot, pl.ds(h + PR, tl), col] = src_next[
                0, pl.ds(0, tl), pl.ds(0, C)].astype(jnp.bfloat16)

    def build_first(slot, src, src_next):
        for kw in range(3):
            col = pl.ds(kw * C, C)
            h = WP + 1 - kw
            ring[slot, pl.ds(0, h), col] = jnp.zeros((h, C), jnp.bfloat16)
            ring[slot, pl.ds(h, PR), col] = src[
                0, pl.ds(0, PR), pl.ds(0, C)].astype(jnp.bfloat16)
            tl = MEXT - PR - h
            ring[slot, pl.ds(h + PR, tl), col] = src_next[
                0, pl.ds(0, tl), pl.ds(0, C)].astype(jnp.bfloat16)

    def matmuls():
        for m0, mlen, r0, rlen in ((0, MHALF + 2 * WP, 0, MHALF),
                                   (MHALF, MEXT - MHALF, MHALF, PR - MHALF)):
            for kd, slot in enumerate((s0, s1, s2)):
                x = ring[slot, pl.ds(m0, mlen), :]
                if kd == 0:
                    acc3[pl.ds(0, mlen), :] = jnp.dot(
                        x, wbig[kd], preferred_element_type=jnp.float32)
                else:
                    acc3[pl.ds(0, mlen), :] += jnp.dot(
                        x, wbig[kd], preferred_element_type=jnp.float32)
            res = (acc3[pl.ds(0, rlen), pl.ds(0, C)]
                   + acc3[pl.ds(WP, rlen), pl.ds(C, C)]
                   + acc3[pl.ds(2 * WP, rlen), pl.ds(2 * C, C)])
            msk = in1[0, pl.ds(r0, rlen), C:].astype(jnp.float32)
            res = res * _bn_scale(g) + bt[...]
            res = jnp.maximum(res, 0.0) * msk
            out[0, pl.ds(r0, rlen), :C] = res.astype(jnp.bfloat16)
            out[0, pl.ds(r0, rlen), C:] = msk.astype(jnp.bfloat16)

    is_first = t == first
    if pads is None:
        is_pad = None

        @pl.when(is_first)
        def _():
            build_zero_plane(s0, in1)
            build_first(s1, in1, in2)
            build_main(s2, in1, in2)
            matmuls()

        @pl.when(jnp.logical_not(is_first))
        def _():
            patch_tail(s1, in2)
            build_main(s2, in1, in2)
            matmuls()
    else:
        plo, phi = pads
        is_pad = (t == plo) | (t == phi)

        @pl.when(is_first)
        def _():
            build_zero_plane(s0, in1)
            build_first(s1, in1, in2)
            build_main(s2, in1, in2)
            matmuls()

        @pl.when(jnp.logical_not(is_first) & jnp.logical_not(is_pad))
        def _():
            patch_tail(s1, in2)
            build_main(s2, in1, in2)
            matmuls()

        @pl.when(is_pad)
        def _():
            out[0] = jnp.zeros((PR, C2), jnp.bfloat16)


def _make_sub3(ppb, lo, hi, pads):
    if pads is None:
        per_b = hi - lo + 1
        first = 0

        def pmap(i, dk):
            return (i // per_b) * ppb + (i % per_b) + lo + dk - 1

        def omap(i):
            return ((i // per_b) * ppb + (i % per_b) + lo, 0, 0)

        body_pads = None
    else:
        per_b = ppb
        first = 1

        def pmap(i, dk):
            return ((i // per_b) * ppb
                    + jnp.clip((i % per_b) + dk - 1, 0, ppb - 1))

        def omap(i):
            return ((i // per_b) * ppb + (i % per_b), 0, 0)

        body_pads = (pads[0], pads[1])

    body = functools.partial(_sub3_body, per_b=per_b, first=first,
                             pads=body_pads)

    def run(x, wbig, g, bt):
        return pl.pallas_call(
            body,
            grid=(B * per_b,),
            in_specs=[
                pl.BlockSpec((1, PR, C2), lambda i: (pmap(i, 1), 0, 0)),
                pl.BlockSpec((1, PR, C2), lambda i: (pmap(i, 2), 0, 0)),
                pl.BlockSpec((3, 3 * C, 3 * C), lambda i: (0, 0, 0)),
                pl.BlockSpec((1, C), lambda i: (0, 0)),
                pl.BlockSpec((1, C), lambda i: (0, 0)),
            ],
            out_specs=pl.BlockSpec((1, PR, C2), omap),
            out_shape=jax.ShapeDtypeStruct((x.shape[0], PR, C2), jnp.bfloat16),
            scratch_shapes=[
                pltpu.VMEM((3, MEXT, 3 * C), jnp.bfloat16),
                pltpu.VMEM((MHALF + 2 * WP, 3 * C), jnp.float32),
            ],
        )(x, x, wbig, g, bt)

    return run


def _down2_body(in0, in1, in2, w3k, g, bt, out):
    i = pl.program_id(0)
    is_pad = (i % P2 == 0) | (i % P2 == P2 - 1)

    @pl.when(jnp.logical_not(is_pad))
    def _():
        acc = jnp.dot(in0[0, :, :C], w3k[0], preferred_element_type=jnp.float32)
        acc += jnp.dot(in1[0, :, :C], w3k[1], preferred_element_type=jnp.float32)
        acc += jnp.dot(in2[0, :, :C], w3k[2], preferred_element_type=jnp.float32)
        msum = (in0[0, :, C:].astype(jnp.float32)
                + in1[0, :, C:].astype(jnp.float32)
                + in2[0, :, C:].astype(jnp.float32))
        m = (msum > 0.0).astype(jnp.float32)
        res = acc * _bn_scale(g) + bt[...]
        res = jnp.maximum(res, 0.0) * m
        out[0, :, :C] = res.astype(jnp.bfloat16)
        out[0, :, C:] = m.astype(jnp.bfloat16)

    @pl.when(is_pad)
    def _():
        out[0] = jnp.zeros((PR, C2), jnp.bfloat16)


def _down_conv2(dense, w3k, g, bt):
    def in_map(dk):
        def f(i):
            b = i // P2
            p = jnp.clip(2 * (i % P2) - 1 + dk, 0, P1 - 1)
            return (b * P1 + p, 0, 0)
        return f

    return pl.pallas_call(
        _down2_body,
        grid=(B * P2,),
        in_specs=[
            pl.BlockSpec((1, PR, C2), in_map(0)),
            pl.BlockSpec((1, PR, C2), in_map(1)),
            pl.BlockSpec((1, PR, C2), in_map(2)),
            pl.BlockSpec((3, C, C), lambda i: (0, 0, 0)),
            pl.BlockSpec((1, C), lambda i: (0, 0)),
            pl.BlockSpec((1, C), lambda i: (0, 0)),
        ],
        out_specs=pl.BlockSpec((1, PR, C2), lambda i: (i, 0, 0)),
        out_shape=jax.ShapeDtypeStruct((NP2, PR, C2), jnp.bfloat16),
    )(dense, dense, dense, w3k, g, bt)


def _down5_body(in0, in1, in2, w3k, g, bt, out):
    acc = jnp.dot(in0[0, :, :C], w3k[0], preferred_element_type=jnp.float32)
    acc += jnp.dot(in1[0, :, :C], w3k[1], preferred_element_type=jnp.float32)
    acc += jnp.dot(in2[0, :, :C], w3k[2], preferred_element_type=jnp.float32)
    msum = (in0[0, :, C:].astype(jnp.float32)
            + in1[0, :, C:].astype(jnp.float32)
            + in2[0, :, C:].astype(jnp.float32))
    m = (msum > 0.0).astype(jnp.float32)
    res = acc * _bn_scale(g) + bt[...]
    out[0] = jnp.maximum(res, 0.0) * m


def _down_conv5(h4, w3k, g, bt):
    def in_map(dk):
        def f(i):
            b = i // D3
            return (b * P2 + 2 * (i % D3) + 1 + dk, 0, 0)
        return f

    return pl.pallas_call(
        _down5_body,
        grid=(B * D3,),
        in_specs=[
            pl.BlockSpec((1, PR, C2), in_map(0)),
            pl.BlockSpec((1, PR, C2), in_map(1)),
            pl.BlockSpec((1, PR, C2), in_map(2)),
            pl.BlockSpec((3, C, C), lambda i: (0, 0, 0)),
            pl.BlockSpec((1, C), lambda i: (0, 0)),
            pl.BlockSpec((1, C), lambda i: (0, 0)),
        ],
        out_specs=pl.BlockSpec((1, PR, C), lambda i: (i, 0, 0)),
        out_shape=jax.ShapeDtypeStruct((B * D3, PR, C), jnp.float32),
    )(h4, h4, h4, w3k, g, bt)


def _wbig(w):
    return jnp.transpose(w, (0, 2, 3, 1, 4)).reshape(3, 3 * C, 3 * C).astype(
        jnp.bfloat16)


def _conv_stack(dense, w1, w2, w3, w4, w5,
                g1, g2, g3, g4, g5, bt1, bt2, bt3, bt4, bt5):
    gs = [x.reshape(1, C) for x in (g1, g2, g3, g4, g5)]
    bts = [x.reshape(1, C) for x in (bt1, bt2, bt3, bt4, bt5)]

    sub1 = _make_sub3(P1, 1, D, None)
    h1 = sub1(dense, _wbig(w1), gs[0], bts[0])
    h2 = _down_conv2(h1, w2.reshape(3, C, C).astype(jnp.bfloat16),
                     gs[1], bts[1])
    sub3 = _make_sub3(P2, 0, 0, (0, P2 - 1))
    h3 = sub3(h2, _wbig(w3), gs[2], bts[2])
    sub4 = _make_sub3(P2, 1, D2, None)
    h4 = sub4(h3, _wbig(w4), gs[3], bts[3])
    h5 = _down_conv5(h4, w5.reshape(3, C, C).astype(jnp.bfloat16),
                     gs[4], bts[4])
    out = h5.reshape(B, D3, HP, WP, C)[:, :, 1:97, 1:97, :]
    return jnp.transpose(out, (0, 4, 1, 2, 3))


def kernel(features, coords, w1, w2, w3, w4, w5,
           g1, g2, g3, g4, g5, bt1, bt2, bt3, bt4, bt5):
    feats128 = jnp.pad(jnp.concatenate(
        [features, jnp.ones((N, C), jnp.float32)], axis=1),
        ((0, NPAD - N), (0, 0)))
    cpad = jnp.pad(coords, ((0, NPAD - N), (0, 0)))
    dense = _sc_scatter(feats128, cpad[:, 0], cpad[:, 1], cpad[:, 2],
                        cpad[:, 3])
    return _conv_stack(dense, w1, w2, w3, w4, w5,
                       g1, g2, g3, g4, g5, bt1, bt2, bt3, bt4, bt5)
